# baseline (speedup 1.0000x reference)
import functools

import jax
import jax.numpy as jnp
import numpy as np
from jax import lax
from jax.experimental import pallas as pl
from jax.experimental.pallas import tpu as pltpu
from jax.experimental.pallas import tpu_sc as plsc

N = 10000
E = 320000
D = 128

NC = 2
NS = 16
NW = NC * NS
EPW = E // NW
CHUNK = 128
NFULL = EPW // CHUNK
TAIL = EPW - NFULL * CHUNK
NPAD = 10112
RPT = NPAD // NS
ZR = 8
NZ = RPT // ZR

BN = 1000
BE = 2000
GE = E // BE

_SCALE = float(1.0 / np.sqrt(D))


def _qkv_body(cur_ref, x_ref, w_ref, b_ref, o_ref):
    c = jnp.abs(cur_ref[0, 0])
    sqc = jnp.sqrt(c)
    x = x_ref[...]
    nrm = jnp.sqrt(jnp.sum(x * x, axis=-1, keepdims=True))
    nrm = jnp.maximum(nrm, 1e-10)
    s = jnp.clip(sqc * nrm, 0.0, 1.0 - 1e-5)
    at = 0.5 * jnp.log((1.0 + s) / (1.0 - s))
    xt = at * x / (sqc * nrm)
    o_ref[...] = lax.dot_general(
        xt, w_ref[...], (((1,), (1,)), ((), ())),
        preferred_element_type=jnp.float32) + b_ref[...]


def _qkv(cur, x, w, b2):
    return pl.pallas_call(
        _qkv_body,
        grid=(N // BN,),
        in_specs=[
            pl.BlockSpec((1, 1), lambda i: (0, 0)),
            pl.BlockSpec((BN, D), lambda i: (i, 0)),
            pl.BlockSpec((3 * D, D), lambda i: (0, 0)),
            pl.BlockSpec((1, 3 * D), lambda i: (0, 0)),
        ],
        out_specs=pl.BlockSpec((BN, 3 * D), lambda i: (i, 0)),
        out_shape=jax.ShapeDtypeStruct((N, 3 * D), jnp.float32),
    )(cur, x, w, b2)


_sc_mesh = plsc.VectorSubcoreMesh(core_axis_name="c", subcore_axis_name="s")


@functools.partial(
    pl.kernel,
    mesh=_sc_mesh,
    out_type=[
        jax.ShapeDtypeStruct((E, D), jnp.float32),
        jax.ShapeDtypeStruct((E, D), jnp.float32),
        jax.ShapeDtypeStruct((E, D), jnp.float32),
    ],
    scratch_types=[
        pltpu.VMEM((CHUNK,), jnp.int32),
        pltpu.VMEM((CHUNK,), jnp.int32),
        pltpu.VMEM((CHUNK,), jnp.int32),
        pltpu.VMEM((CHUNK,), jnp.int32),
        pltpu.VMEM((CHUNK, D), jnp.float32),
        pltpu.VMEM((CHUNK, D), jnp.float32),
        pltpu.VMEM((CHUNK, D), jnp.float32),
        pltpu.VMEM((CHUNK, D), jnp.float32),
        pltpu.VMEM((CHUNK, D), jnp.float32),
        pltpu.VMEM((CHUNK, D), jnp.float32),
        pltpu.VMEM((TAIL,), jnp.int32),
        pltpu.VMEM((TAIL,), jnp.int32),
        pltpu.VMEM((TAIL, D), jnp.float32),
        pltpu.SemaphoreType.DMA,
        pltpu.SemaphoreType.DMA,
        pltpu.SemaphoreType.DMA,
        pltpu.SemaphoreType.DMA,
        pltpu.SemaphoreType.DMA,
        pltpu.SemaphoreType.DMA,
        pltpu.SemaphoreType.DMA,
    ],
)
def _sc_gather(q_hbm, k_hbm, v_hbm, dst_hbm, src_hbm,
               qd_hbm, ks_hbm, vs_hbm,
               dsti0, dsti1, srci0, srci1,
               rq0, rq1, rk0, rk1, rv0, rv1,
               dsti_t, srci_t, rows_t,
               si0, si1, sg0, sg1, sw0, sw1, st):
    wid = lax.axis_index("s") * NC + lax.axis_index("c")
    dsti = (dsti0, dsti1)
    srci = (srci0, srci1)
    rq = (rq0, rq1)
    rk = (rk0, rk1)
    rv = (rv0, rv1)
    si = (si0, si1)
    sg = (sg0, sg1)
    sw = (sw0, sw1)

    def ebase(i):
        return pl.multiple_of(wid * EPW + i * CHUNK, 8)

    b0 = ebase(0)
    pltpu.make_async_copy(dst_hbm.at[pl.ds(b0, CHUNK)], dsti[0], si[0]).start()
    pltpu.make_async_copy(src_hbm.at[pl.ds(b0, CHUNK)], srci[0], si[0]).start()

    def pair(jj, _):
        for b in range(2):
            i = jj * 2 + b
            base = ebase(i)

            @pl.when(jj >= 1)
            def _():
                bm2 = ebase(i - 2)
                pltpu.make_async_copy(rq[b], qd_hbm.at[pl.ds(bm2, CHUNK)], sw[b]).wait()
                pltpu.make_async_copy(rk[b], ks_hbm.at[pl.ds(bm2, CHUNK)], sw[b]).wait()
                pltpu.make_async_copy(rv[b], vs_hbm.at[pl.ds(bm2, CHUNK)], sw[b]).wait()

            pltpu.make_async_copy(dst_hbm.at[pl.ds(base, CHUNK)], dsti[b], si[b]).wait()
            pltpu.make_async_copy(src_hbm.at[pl.ds(base, CHUNK)], srci[b], si[b]).wait()

            pltpu.make_async_copy(q_hbm.at[dsti[b]], rq[b], sg[b]).start()
            pltpu.make_async_copy(k_hbm.at[srci[b]], rk[b], sg[b]).start()
            pltpu.make_async_copy(v_hbm.at[srci[b]], rv[b], sg[b]).start()

            @pl.when(i < NFULL - 1)
            def _():
                bn = ebase(i + 1)
                pltpu.make_async_copy(dst_hbm.at[pl.ds(bn, CHUNK)], dsti[1 - b], si[1 - b]).start()
                pltpu.make_async_copy(src_hbm.at[pl.ds(bn, CHUNK)], srci[1 - b], si[1 - b]).start()

            pltpu.make_async_copy(q_hbm.at[dsti[b]], rq[b], sg[b]).wait()
            pltpu.make_async_copy(k_hbm.at[srci[b]], rk[b], sg[b]).wait()
            pltpu.make_async_copy(v_hbm.at[srci[b]], rv[b], sg[b]).wait()

            pltpu.make_async_copy(rq[b], qd_hbm.at[pl.ds(base, CHUNK)], sw[b]).start()
            pltpu.make_async_copy(rk[b], ks_hbm.at[pl.ds(base, CHUNK)], sw[b]).start()
            pltpu.make_async_copy(rv[b], vs_hbm.at[pl.ds(base, CHUNK)], sw[b]).start()
        return 0

    lax.fori_loop(0, NFULL // 2, pair, 0)

    for b in range(2):
        bl = ebase(NFULL - 2 + b)
        pltpu.make_async_copy(rq[b], qd_hbm.at[pl.ds(bl, CHUNK)], sw[b]).wait()
        pltpu.make_async_copy(rk[b], ks_hbm.at[pl.ds(bl, CHUNK)], sw[b]).wait()
        pltpu.make_async_copy(rv[b], vs_hbm.at[pl.ds(bl, CHUNK)], sw[b]).wait()

    tb = pl.multiple_of(wid * EPW + NFULL * CHUNK, 8)
    pltpu.sync_copy(dst_hbm.at[pl.ds(tb, TAIL)], dsti_t)
    pltpu.sync_copy(src_hbm.at[pl.ds(tb, TAIL)], srci_t)
    pltpu.async_copy(q_hbm.at[dsti_t], rows_t, st).wait()
    pltpu.sync_copy(rows_t, qd_hbm.at[pl.ds(tb, TAIL)])
    pltpu.async_copy(k_hbm.at[srci_t], rows_t, st).wait()
    pltpu.sync_copy(rows_t, ks_hbm.at[pl.ds(tb, TAIL)])
    pltpu.async_copy(v_hbm.at[srci_t], rows_t, st).wait()
    pltpu.sync_copy(rows_t, vs_hbm.at[pl.ds(tb, TAIL)])


def _scores_body(qd_ref, ks_ref, s_ref, m_ref):
    i = pl.program_id(0)
    s = jnp.sum(qd_ref[...] * ks_ref[...], axis=-1, keepdims=True) * _SCALE
    s_ref[...] = s
    bm = jnp.max(s)

    @pl.when(i == 0)
    def _():
        m_ref[...] = jnp.reshape(bm, (1, 1))

    @pl.when(i > 0)
    def _():
        m_ref[...] = jnp.maximum(m_ref[...], bm)


def _scores(qd, ks):
    return pl.pallas_call(
        _scores_body,
        grid=(GE,),
        in_specs=[
            pl.BlockSpec((BE, D), lambda i: (i, 0)),
            pl.BlockSpec((BE, D), lambda i: (i, 0)),
        ],
        out_specs=[
            pl.BlockSpec((BE, 1), lambda i: (i, 0)),
            pl.BlockSpec((1, 1), lambda i: (0, 0)),
        ],
        out_shape=[
            jax.ShapeDtypeStruct((E, 1), jnp.float32),
            jax.ShapeDtypeStruct((1, 1), jnp.float32),
        ],
    )(qd, ks)


def _weights_body(s_ref, m_ref, vs_ref, prod_ref):
    ex = jnp.exp(s_ref[...] - m_ref[0, 0])
    prod_ref[...] = ex * vs_ref[...]


def _weights(s_col, m, vs):
    return pl.pallas_call(
        _weights_body,
        grid=(GE,),
        in_specs=[
            pl.BlockSpec((BE, 1), lambda i: (i, 0)),
            pl.BlockSpec((1, 1), lambda i: (0, 0)),
            pl.BlockSpec((BE, D), lambda i: (i, 0)),
        ],
        out_specs=pl.BlockSpec((BE, D), lambda i: (i, 0)),
        out_shape=jax.ShapeDtypeStruct((E, D), jnp.float32),
    )(s_col, m, vs)


SEG = NPAD // D
SEGP = 80


def _denom_body(s_ref, m_ref, dst_ref, den_ref):
    i = pl.program_id(0)
    ex = jnp.exp(s_ref[...] - m_ref[0, 0])
    dstv = dst_ref[...]
    col = jnp.remainder(dstv, D)
    seg = dstv // D
    lane = lax.broadcasted_iota(jnp.int32, (BE, D), 1)
    exmat = jnp.where(lane == col, ex, 0.0)
    seglane = lax.broadcasted_iota(jnp.int32, (BE, SEGP), 1)
    oh = jnp.where(seglane == seg, 1.0, 0.0)
    contrib = lax.dot_general(oh, exmat, (((0,), (0,)), ((), ())),
                              preferred_element_type=jnp.float32)

    @pl.when(i == 0)
    def _():
        den_ref[...] = contrib

    @pl.when(i > 0)
    def _():
        den_ref[...] = den_ref[...] + contrib


def _denom(s_col, m, dst2):
    return pl.pallas_call(
        _denom_body,
        grid=(GE,),
        in_specs=[
            pl.BlockSpec((BE, 1), lambda i: (i, 0)),
            pl.BlockSpec((1, 1), lambda i: (0, 0)),
            pl.BlockSpec((BE, 1), lambda i: (i, 0)),
        ],
        out_specs=pl.BlockSpec((SEGP, D), lambda i: (0, 0)),
        out_shape=jax.ShapeDtypeStruct((SEGP, D), jnp.float32),
    )(s_col, m, dst2)


@functools.partial(
    pl.kernel,
    mesh=_sc_mesh,
    out_type=[
        jax.ShapeDtypeStruct((NC, NPAD, D), jnp.float32),
    ],
    scratch_types=[
        pltpu.VMEM_SHARED((NPAD, D), jnp.float32),
        pltpu.VMEM((CHUNK,), jnp.int32),
        pltpu.VMEM((CHUNK,), jnp.int32),
        pltpu.VMEM((CHUNK, D), jnp.float32),
        pltpu.VMEM((CHUNK, D), jnp.float32),
        pltpu.VMEM((TAIL,), jnp.int32),
        pltpu.VMEM((TAIL, D), jnp.float32),
        pltpu.VMEM((ZR, D), jnp.float32),
        pltpu.SemaphoreType.DMA,
        pltpu.SemaphoreType.DMA,
        pltpu.SemaphoreType.DMA,
        pltpu.SemaphoreType.DMA,
    ],
)
def _sc_scatter(prod_hbm, dst_hbm,
                acc_out,
                accum, idxb0, idxb1, rows0, rows1, idxt, rowst, zb,
                sl0, sl1, sc0, sc1):
    cid = lax.axis_index("c")
    sid = lax.axis_index("s")
    wid = sid * NC + cid

    zeros16 = jnp.zeros((16,), jnp.float32)

    def zb_body(i, _):
        r = i // 8
        col = (i % 8) * 16
        zb[r, pl.ds(col, 16)] = zeros16
        return 0

    lax.fori_loop(0, ZR * 8, zb_body, 0)

    def zc_body(j, _):
        r0 = pl.multiple_of(sid * RPT + j * ZR, 8)
        pltpu.sync_copy(zb, accum.at[pl.ds(r0, ZR)])
        return 0

    lax.fori_loop(0, NZ, zc_body, 0)
    plsc.subcore_barrier()

    idxb = (idxb0, idxb1)
    rows = (rows0, rows1)
    sl = (sl0, sl1)
    sc = (sc0, sc1)

    def ebase(i):
        return pl.multiple_of(wid * EPW + i * CHUNK, 8)

    b0 = ebase(0)
    pltpu.make_async_copy(dst_hbm.at[pl.ds(b0, CHUNK)], idxb[0], sl[0]).start()
    pltpu.make_async_copy(prod_hbm.at[pl.ds(b0, CHUNK)], rows[0], sl[0]).start()

    def pair(jj, _):
        for b in range(2):
            i = jj * 2 + b
            base = ebase(i)
            pltpu.make_async_copy(dst_hbm.at[pl.ds(base, CHUNK)], idxb[b], sl[b]).wait()
            pltpu.make_async_copy(prod_hbm.at[pl.ds(base, CHUNK)], rows[b], sl[b]).wait()
            pltpu.async_copy(rows[b], accum.at[idxb[b]], sc[b], add=True)

            @pl.when(i >= 1)
            def _():
                pltpu.make_async_copy(rows[1 - b], accum.at[idxb[1 - b]], sc[1 - b]).wait()

            @pl.when(i < NFULL - 1)
            def _():
                bn = ebase(i + 1)
                pltpu.make_async_copy(dst_hbm.at[pl.ds(bn, CHUNK)], idxb[1 - b], sl[1 - b]).start()
                pltpu.make_async_copy(prod_hbm.at[pl.ds(bn, CHUNK)], rows[1 - b], sl[1 - b]).start()
        return 0

    lax.fori_loop(0, NFULL // 2, pair, 0)
    pltpu.make_async_copy(rows[1], accum.at[idxb[1]], sc[1]).wait()

    tb = pl.multiple_of(wid * EPW + NFULL * CHUNK, 8)
    pltpu.sync_copy(dst_hbm.at[pl.ds(tb, TAIL)], idxt)
    pltpu.sync_copy(prod_hbm.at[pl.ds(tb, TAIL)], rowst)
    pltpu.sync_copy(rowst, accum.at[idxt], add=True)

    plsc.subcore_barrier()

    def dump(j, _):
        r0 = pl.multiple_of(sid * RPT + j * ZR, 8)
        pltpu.sync_copy(accum.at[pl.ds(r0, ZR)], zb)
        pltpu.sync_copy(zb, acc_out.at[cid, pl.ds(r0, ZR)])
        return 0

    lax.fori_loop(0, NZ, dump, 0)


def _final_body(cur_ref, pa_ref, da_ref, v_ref, w_ref, b_ref, o_ref):
    agg = pa_ref[0] + pa_ref[1]
    den = da_ref[...]
    agg = agg / jnp.maximum(den, 1e-12)
    agg = jnp.where(den > 0.0, agg, v_ref[...])
    h = lax.dot_general(
        agg, w_ref[...], (((1,), (1,)), ((), ())),
        preferred_element_type=jnp.float32) + b_ref[...]
    c = jnp.abs(cur_ref[0, 0])
    sqc = jnp.sqrt(c)
    nrm = jnp.sqrt(jnp.sum(h * h, axis=-1, keepdims=True))
    nrm = jnp.maximum(nrm, 1e-10)
    o_ref[...] = jnp.tanh(sqc * nrm) * h / (sqc * nrm)


def _final(cur, acc, den, v, w, b2):
    return pl.pallas_call(
        _final_body,
        grid=(N // BN,),
        in_specs=[
            pl.BlockSpec((1, 1), lambda i: (0, 0)),
            pl.BlockSpec((NC, BN, D), lambda i: (0, i, 0)),
            pl.BlockSpec((BN, 1), lambda i: (i, 0)),
            pl.BlockSpec((BN, D), lambda i: (i, 0)),
            pl.BlockSpec((D, D), lambda i: (0, 0)),
            pl.BlockSpec((1, D), lambda i: (0, 0)),
        ],
        out_specs=pl.BlockSpec((BN, D), lambda i: (i, 0)),
        out_shape=jax.ShapeDtypeStruct((N, D), jnp.float32),
    )(cur, acc, den, v, w, b2)


def kernel(node_embeddings, curvature, in_w0, in_b0, out_w0, out_b0,
           in_w1, in_b1, out_w1, out_b1, edge_index):
    cur = jnp.reshape(curvature, (1, 1)).astype(jnp.float32)
    src = edge_index[0]
    dst = edge_index[1]
    x = node_embeddings
    for (iw, ib, ow, ob) in ((in_w0, in_b0, out_w0, out_b0),
                             (in_w1, in_b1, out_w1, out_b1)):
        qkv = _qkv(cur, x, iw, jnp.reshape(ib, (1, 3 * D)))
        q = qkv[:, :D]
        k = qkv[:, D:2 * D]
        v = qkv[:, 2 * D:]
        qd, ks, vs = _sc_gather(q, k, v, dst, src)
        s_col, m = _scores(qd, ks)
        prod = _weights(s_col, m, vs)
        denmat = _denom(s_col, m, jnp.reshape(dst, (E, 1)))
        den = jnp.reshape(denmat, (SEGP * D, 1))[:N]
        (acc,) = _sc_scatter(prod, dst)
        x = _final(cur, acc, den, v, ow, jnp.reshape(ob, (1, D)))
    return x


# speedup vs baseline: 1.0085x; 1.0085x over previous
import functools

import jax
import jax.numpy as jnp
import numpy as np
from jax import lax
from jax.experimental import pallas as pl
from jax.experimental.pallas import tpu as pltpu
from jax.experimental.pallas import tpu_sc as plsc

N = 10000
E = 320000
D = 128

NC = 2
NS = 16
NW = NC * NS
EPW = E // NW
CHUNK = 128
NFULL = EPW // CHUNK
TAIL = EPW - NFULL * CHUNK
NPAD = 10112
RPT = NPAD // NS
ZR = 8
NZ = RPT // ZR

BN = 1000
BE = 2000
GE = E // BE

_SCALE = float(1.0 / np.sqrt(D))


def _qkv_body(cur_ref, x_ref, w_ref, b_ref, o_ref):
    c = jnp.abs(cur_ref[0, 0])
    sqc = jnp.sqrt(c)
    x = x_ref[...]
    nrm = jnp.sqrt(jnp.sum(x * x, axis=-1, keepdims=True))
    nrm = jnp.maximum(nrm, 1e-10)
    s = jnp.clip(sqc * nrm, 0.0, 1.0 - 1e-5)
    at = 0.5 * jnp.log((1.0 + s) / (1.0 - s))
    xt = at * x / (sqc * nrm)
    o_ref[...] = lax.dot_general(
        xt, w_ref[...], (((1,), (1,)), ((), ())),
        preferred_element_type=jnp.float32) + b_ref[...]


def _qkv(cur, x, w, b2):
    return pl.pallas_call(
        _qkv_body,
        grid=(N // BN,),
        in_specs=[
            pl.BlockSpec((1, 1), lambda i: (0, 0)),
            pl.BlockSpec((BN, D), lambda i: (i, 0)),
            pl.BlockSpec((3 * D, D), lambda i: (0, 0)),
            pl.BlockSpec((1, 3 * D), lambda i: (0, 0)),
        ],
        out_specs=pl.BlockSpec((BN, 3 * D), lambda i: (i, 0)),
        out_shape=jax.ShapeDtypeStruct((N, 3 * D), jnp.float32),
    )(cur, x, w, b2)


_sc_mesh = plsc.VectorSubcoreMesh(core_axis_name="c", subcore_axis_name="s")


@functools.partial(
    pl.kernel,
    mesh=_sc_mesh,
    out_type=[
        jax.ShapeDtypeStruct((E, D), jnp.float32),
        jax.ShapeDtypeStruct((E, D), jnp.float32),
        jax.ShapeDtypeStruct((E, D), jnp.float32),
    ],
    scratch_types=[
        pltpu.VMEM((CHUNK,), jnp.int32),
        pltpu.VMEM((CHUNK,), jnp.int32),
        pltpu.VMEM((CHUNK,), jnp.int32),
        pltpu.VMEM((CHUNK,), jnp.int32),
        pltpu.VMEM((CHUNK, D), jnp.float32),
        pltpu.VMEM((CHUNK, D), jnp.float32),
        pltpu.VMEM((CHUNK, D), jnp.float32),
        pltpu.VMEM((CHUNK, D), jnp.float32),
        pltpu.VMEM((CHUNK, D), jnp.float32),
        pltpu.VMEM((CHUNK, D), jnp.float32),
        pltpu.VMEM((TAIL,), jnp.int32),
        pltpu.VMEM((TAIL,), jnp.int32),
        pltpu.VMEM((TAIL, D), jnp.float32),
        pltpu.SemaphoreType.DMA,
        pltpu.SemaphoreType.DMA,
        pltpu.SemaphoreType.DMA,
        pltpu.SemaphoreType.DMA,
        pltpu.SemaphoreType.DMA,
        pltpu.SemaphoreType.DMA,
        pltpu.SemaphoreType.DMA,
    ],
)
def _sc_gather(q_hbm, k_hbm, v_hbm, dst_hbm, src_hbm,
               qd_hbm, ks_hbm, vs_hbm,
               dsti0, dsti1, srci0, srci1,
               rq0, rq1, rk0, rk1, rv0, rv1,
               dsti_t, srci_t, rows_t,
               si0, si1, sg0, sg1, sw0, sw1, st):
    wid = lax.axis_index("s") * NC + lax.axis_index("c")
    dsti = (dsti0, dsti1)
    srci = (srci0, srci1)
    rq = (rq0, rq1)
    rk = (rk0, rk1)
    rv = (rv0, rv1)
    si = (si0, si1)
    sg = (sg0, sg1)
    sw = (sw0, sw1)

    def ebase(i):
        return pl.multiple_of(wid * EPW + i * CHUNK, 8)

    b0 = ebase(0)
    pltpu.make_async_copy(dst_hbm.at[pl.ds(b0, CHUNK)], dsti[0], si[0]).start()
    pltpu.make_async_copy(src_hbm.at[pl.ds(b0, CHUNK)], srci[0], si[0]).start()

    def pair(jj, _):
        for b in range(2):
            i = jj * 2 + b
            base = ebase(i)

            @pl.when(jj >= 1)
            def _():
                bm2 = ebase(i - 2)
                pltpu.make_async_copy(rq[b], qd_hbm.at[pl.ds(bm2, CHUNK)], sw[b]).wait()
                pltpu.make_async_copy(rk[b], ks_hbm.at[pl.ds(bm2, CHUNK)], sw[b]).wait()
                pltpu.make_async_copy(rv[b], vs_hbm.at[pl.ds(bm2, CHUNK)], sw[b]).wait()

            pltpu.make_async_copy(dst_hbm.at[pl.ds(base, CHUNK)], dsti[b], si[b]).wait()
            pltpu.make_async_copy(src_hbm.at[pl.ds(base, CHUNK)], srci[b], si[b]).wait()

            pltpu.make_async_copy(q_hbm.at[dsti[b]], rq[b], sg[b]).start()
            pltpu.make_async_copy(k_hbm.at[srci[b]], rk[b], sg[b]).start()
            pltpu.make_async_copy(v_hbm.at[srci[b]], rv[b], sg[b]).start()

            @pl.when(i < NFULL - 1)
            def _():
                bn = ebase(i + 1)
                pltpu.make_async_copy(dst_hbm.at[pl.ds(bn, CHUNK)], dsti[1 - b], si[1 - b]).start()
                pltpu.make_async_copy(src_hbm.at[pl.ds(bn, CHUNK)], srci[1 - b], si[1 - b]).start()

            pltpu.make_async_copy(q_hbm.at[dsti[b]], rq[b], sg[b]).wait()
            pltpu.make_async_copy(k_hbm.at[srci[b]], rk[b], sg[b]).wait()
            pltpu.make_async_copy(v_hbm.at[srci[b]], rv[b], sg[b]).wait()

            pltpu.make_async_copy(rq[b], qd_hbm.at[pl.ds(base, CHUNK)], sw[b]).start()
            pltpu.make_async_copy(rk[b], ks_hbm.at[pl.ds(base, CHUNK)], sw[b]).start()
            pltpu.make_async_copy(rv[b], vs_hbm.at[pl.ds(base, CHUNK)], sw[b]).start()
        return 0

    lax.fori_loop(0, NFULL // 2, pair, 0)

    for b in range(2):
        bl = ebase(NFULL - 2 + b)
        pltpu.make_async_copy(rq[b], qd_hbm.at[pl.ds(bl, CHUNK)], sw[b]).wait()
        pltpu.make_async_copy(rk[b], ks_hbm.at[pl.ds(bl, CHUNK)], sw[b]).wait()
        pltpu.make_async_copy(rv[b], vs_hbm.at[pl.ds(bl, CHUNK)], sw[b]).wait()

    tb = pl.multiple_of(wid * EPW + NFULL * CHUNK, 8)
    pltpu.sync_copy(dst_hbm.at[pl.ds(tb, TAIL)], dsti_t)
    pltpu.sync_copy(src_hbm.at[pl.ds(tb, TAIL)], srci_t)
    pltpu.async_copy(q_hbm.at[dsti_t], rows_t, st).wait()
    pltpu.sync_copy(rows_t, qd_hbm.at[pl.ds(tb, TAIL)])
    pltpu.async_copy(k_hbm.at[srci_t], rows_t, st).wait()
    pltpu.sync_copy(rows_t, ks_hbm.at[pl.ds(tb, TAIL)])
    pltpu.async_copy(v_hbm.at[srci_t], rows_t, st).wait()
    pltpu.sync_copy(rows_t, vs_hbm.at[pl.ds(tb, TAIL)])


def _scores_body(qd_ref, ks_ref, s_ref, m_ref):
    i = pl.program_id(0)
    s = jnp.sum(qd_ref[...] * ks_ref[...], axis=-1, keepdims=True) * _SCALE
    s_ref[...] = s
    bm = jnp.max(s)

    @pl.when(i == 0)
    def _():
        m_ref[...] = jnp.reshape(bm, (1, 1))

    @pl.when(i > 0)
    def _():
        m_ref[...] = jnp.maximum(m_ref[...], bm)


def _scores(qd, ks):
    return pl.pallas_call(
        _scores_body,
        grid=(GE,),
        in_specs=[
            pl.BlockSpec((BE, D), lambda i: (i, 0)),
            pl.BlockSpec((BE, D), lambda i: (i, 0)),
        ],
        out_specs=[
            pl.BlockSpec((BE, 1), lambda i: (i, 0)),
            pl.BlockSpec((1, 1), lambda i: (0, 0)),
        ],
        out_shape=[
            jax.ShapeDtypeStruct((E, 1), jnp.float32),
            jax.ShapeDtypeStruct((1, 1), jnp.float32),
        ],
    )(qd, ks)


def _weights_body(s_ref, m_ref, vs_ref, prod_ref):
    ex = jnp.exp(s_ref[...] - m_ref[0, 0])
    prod_ref[...] = ex * vs_ref[...]


def _weights(s_col, m, vs):
    return pl.pallas_call(
        _weights_body,
        grid=(GE,),
        in_specs=[
            pl.BlockSpec((BE, 1), lambda i: (i, 0)),
            pl.BlockSpec((1, 1), lambda i: (0, 0)),
            pl.BlockSpec((BE, D), lambda i: (i, 0)),
        ],
        out_specs=pl.BlockSpec((BE, D), lambda i: (i, 0)),
        out_shape=jax.ShapeDtypeStruct((E, D), jnp.float32),
    )(s_col, m, vs)


SEG = NPAD // D
SEGP = 80


def _denom_body(s_ref, m_ref, dst_ref, den_ref):
    i = pl.program_id(0)
    ex = jnp.exp(s_ref[...] - m_ref[0, 0])
    dstv = dst_ref[...]
    col = jnp.remainder(dstv, D)
    seg = dstv // D
    lane = lax.broadcasted_iota(jnp.int32, (BE, D), 1)
    exmat = jnp.where(lane == col, ex, 0.0)
    seglane = lax.broadcasted_iota(jnp.int32, (BE, SEGP), 1)
    oh = jnp.where(seglane == seg, 1.0, 0.0)
    contrib = lax.dot_general(oh, exmat, (((0,), (0,)), ((), ())),
                              preferred_element_type=jnp.float32)

    @pl.when(i == 0)
    def _():
        den_ref[...] = contrib

    @pl.when(i > 0)
    def _():
        den_ref[...] = den_ref[...] + contrib


def _denom(s_col, m, dst2):
    return pl.pallas_call(
        _denom_body,
        grid=(GE,),
        in_specs=[
            pl.BlockSpec((BE, 1), lambda i: (i, 0)),
            pl.BlockSpec((1, 1), lambda i: (0, 0)),
            pl.BlockSpec((BE, 1), lambda i: (i, 0)),
        ],
        out_specs=pl.BlockSpec((SEGP, D), lambda i: (0, 0)),
        out_shape=jax.ShapeDtypeStruct((SEGP, D), jnp.float32),
    )(s_col, m, dst2)


@functools.partial(
    pl.kernel,
    mesh=_sc_mesh,
    out_type=[
        jax.ShapeDtypeStruct((NC, NPAD, D), jnp.float32),
    ],
    scratch_types=[
        pltpu.VMEM_SHARED((NPAD, D), jnp.float32),
        pltpu.VMEM((CHUNK,), jnp.int32),
        pltpu.VMEM((CHUNK,), jnp.int32),
        pltpu.VMEM((CHUNK, D), jnp.float32),
        pltpu.VMEM((CHUNK, D), jnp.float32),
        pltpu.VMEM((TAIL,), jnp.int32),
        pltpu.VMEM((TAIL, D), jnp.float32),
        pltpu.VMEM((ZR, D), jnp.float32),
        pltpu.SemaphoreType.DMA,
        pltpu.SemaphoreType.DMA,
        pltpu.SemaphoreType.DMA,
        pltpu.SemaphoreType.DMA,
    ],
)
def _sc_scatter(prod_hbm, dst_hbm,
                acc_out,
                accum, idxb0, idxb1, rows0, rows1, idxt, rowst, zb,
                sl0, sl1, sc0, sc1):
    cid = lax.axis_index("c")
    sid = lax.axis_index("s")
    wid = sid * NC + cid

    zeros16 = jnp.zeros((16,), jnp.float32)

    def zb_body(i, _):
        r = i // 8
        col = (i % 8) * 16
        zb[r, pl.ds(col, 16)] = zeros16
        return 0

    lax.fori_loop(0, ZR * 8, zb_body, 0)

    def zc_body(j, _):
        r0 = pl.multiple_of(sid * RPT + j * ZR, 8)
        pltpu.sync_copy(zb, accum.at[pl.ds(r0, ZR)])
        return 0

    lax.fori_loop(0, NZ, zc_body, 0)
    plsc.subcore_barrier()

    def body(i, _):
        base = pl.multiple_of(wid * EPW + i * CHUNK, 8)
        pltpu.sync_copy(dst_hbm.at[pl.ds(base, CHUNK)], idxb0)
        pltpu.sync_copy(prod_hbm.at[pl.ds(base, CHUNK)], rows0)
        pltpu.sync_copy(rows0, accum.at[idxb0], add=True)
        return 0

    lax.fori_loop(0, NFULL, body, 0)

    tb = pl.multiple_of(wid * EPW + NFULL * CHUNK, 8)
    pltpu.sync_copy(dst_hbm.at[pl.ds(tb, TAIL)], idxt)
    pltpu.sync_copy(prod_hbm.at[pl.ds(tb, TAIL)], rowst)
    pltpu.sync_copy(rowst, accum.at[idxt], add=True)

    plsc.subcore_barrier()

    def dump(j, _):
        r0 = pl.multiple_of(sid * RPT + j * ZR, 8)
        pltpu.sync_copy(accum.at[pl.ds(r0, ZR)], zb)
        pltpu.sync_copy(zb, acc_out.at[cid, pl.ds(r0, ZR)])
        return 0

    lax.fori_loop(0, NZ, dump, 0)


def _final_body(cur_ref, pa_ref, da_ref, v_ref, w_ref, b_ref, o_ref):
    agg = pa_ref[0] + pa_ref[1]
    den = da_ref[...]
    agg = agg / jnp.maximum(den, 1e-12)
    agg = jnp.where(den > 0.0, agg, v_ref[...])
    h = lax.dot_general(
        agg, w_ref[...], (((1,), (1,)), ((), ())),
        preferred_element_type=jnp.float32) + b_ref[...]
    c = jnp.abs(cur_ref[0, 0])
    sqc = jnp.sqrt(c)
    nrm = jnp.sqrt(jnp.sum(h * h, axis=-1, keepdims=True))
    nrm = jnp.maximum(nrm, 1e-10)
    o_ref[...] = jnp.tanh(sqc * nrm) * h / (sqc * nrm)


def _final(cur, acc, den, v, w, b2):
    return pl.pallas_call(
        _final_body,
        grid=(N // BN,),
        in_specs=[
            pl.BlockSpec((1, 1), lambda i: (0, 0)),
            pl.BlockSpec((NC, BN, D), lambda i: (0, i, 0)),
            pl.BlockSpec((BN, 1), lambda i: (i, 0)),
            pl.BlockSpec((BN, D), lambda i: (i, 0)),
            pl.BlockSpec((D, D), lambda i: (0, 0)),
            pl.BlockSpec((1, D), lambda i: (0, 0)),
        ],
        out_specs=pl.BlockSpec((BN, D), lambda i: (i, 0)),
        out_shape=jax.ShapeDtypeStruct((N, D), jnp.float32),
    )(cur, acc, den, v, w, b2)


def kernel(node_embeddings, curvature, in_w0, in_b0, out_w0, out_b0,
           in_w1, in_b1, out_w1, out_b1, edge_index):
    cur = jnp.reshape(curvature, (1, 1)).astype(jnp.float32)
    src = edge_index[0]
    dst = edge_index[1]
    x = node_embeddings
    for (iw, ib, ow, ob) in ((in_w0, in_b0, out_w0, out_b0),
                             (in_w1, in_b1, out_w1, out_b1)):
        qkv = _qkv(cur, x, iw, jnp.reshape(ib, (1, 3 * D)))
        q = qkv[:, :D]
        k = qkv[:, D:2 * D]
        v = qkv[:, 2 * D:]
        qd, ks, vs = _sc_gather(q, k, v, dst, src)
        s_col, m = _scores(qd, ks)
        prod = _weights(s_col, m, vs)
        denmat = _denom(s_col, m, jnp.reshape(dst, (E, 1)))
        den = jnp.reshape(denmat, (SEGP * D, 1))[:N]
        (acc,) = _sc_scatter(prod, dst)
        x = _final(cur, acc, den, v, ow, jnp.reshape(ob, (1, D)))
    return x
